# baseline (speedup 1.0000x reference)
import functools

import jax
import jax.numpy as jnp
from jax import lax
from jax.experimental import pallas as pl
from jax.experimental.pallas import tpu as pltpu
from jax.experimental.pallas import tpu_sc as plsc

D = 128
K = 32
L = 16
NC, NS = 2, 16
NW = NC * NS
NB = 4
G = NB * K


NBUF = 2


def _make_gather_sum(n_pad):
  chunk = n_pad // NW
  nsub = chunk // NB
  mesh = plsc.VectorSubcoreMesh(core_axis_name="c", subcore_axis_name="s")

  @functools.partial(
      pl.kernel,
      mesh=mesh,
      out_type=jax.ShapeDtypeStruct((n_pad, D), jnp.float32),
      scratch_types=[
          pltpu.VMEM((nsub + NBUF, G), jnp.int32),
          pltpu.VMEM((NBUF, G, D), jnp.float32),
          pltpu.VMEM((4 * NBUF * NB, D), jnp.float32),
          pltpu.VMEM_SHARED((n_pad, D), jnp.float32),
          pltpu.SemaphoreType.DMA,
          pltpu.SemaphoreType.DMA,
      ],
  )
  def gsum(h_hbm, idx_hbm, out_hbm, idx_v, gbuf, outv, sbuf, sem0, sem1):
    wid = lax.axis_index("c") * NS + lax.axis_index("s")
    sid = lax.axis_index("s")
    sems = (sem0, sem1)
    pltpu.sync_copy(idx_hbm.at[wid], idx_v)

    srows = n_pad // NS
    soff = sid * srows
    pltpu.sync_copy(h_hbm.at[pl.ds(soff, srows)], sbuf.at[pl.ds(soff, srows)])
    plsc.subcore_barrier()

    for b in range(NBUF):
      pltpu.async_copy(sbuf.at[idx_v.at[b]], gbuf.at[b], sems[b])

    def outer(g, carry):
      for half in range(4):
        for b in range(NBUF):
          q = half * NBUF + b
          s = g * (4 * NBUF) + q
          pltpu.make_async_copy(sbuf.at[idx_v.at[s]], gbuf.at[b],
                                sems[b]).wait()
          for n in range(NB):
            row = n * K
            vals = [[gbuf[b, row + r, pl.ds(j * L, L)] for j in range(D // L)]
                    for r in range(4)]
            for j in range(D // L):
              outv[q * NB + n, pl.ds(j * L, L)] = (
                  (vals[0][j] + vals[1][j]) + (vals[2][j] + vals[3][j]))

            def kgroup(kg, carry, n=n, b=b, q=q):
              row = n * K + kg * 4
              vals = [[gbuf[b, row + r, pl.ds(j * L, L)]
                       for j in range(D // L)] for r in range(4)]
              for j in range(D // L):
                plsc.addupdate(outv.at[q * NB + n, pl.ds(j * L, L)],
                               (vals[0][j] + vals[1][j]) +
                               (vals[2][j] + vals[3][j]))
              return carry

            lax.fori_loop(1, K // 4, kgroup, 0)
          pltpu.async_copy(sbuf.at[idx_v.at[s + NBUF]], gbuf.at[b], sems[b])
      rows = 4 * NBUF * NB
      pltpu.sync_copy(outv, out_hbm.at[pl.ds(wid * chunk + g * rows, rows)])
      return carry

    lax.fori_loop(0, nsub // (4 * NBUF), outer, 0)
    for b in range(NBUF):
      pltpu.make_async_copy(sbuf.at[idx_v.at[b]], gbuf.at[b], sems[b]).wait()

  return gsum


def _make_mm_relu(n_pad, bm):
  def body(x_ref, w_ref, b_ref, o_ref):
    y = jnp.dot(x_ref[...], w_ref[...], preferred_element_type=jnp.float32)
    o_ref[...] = jnp.maximum(y * (1.0 / K) + b_ref[...], 0.0)

  return pl.pallas_call(
      body,
      grid=(n_pad // bm,),
      in_specs=[
          pl.BlockSpec((bm, D), lambda i: (i, 0)),
          pl.BlockSpec((D, D), lambda i: (0, 0)),
          pl.BlockSpec((1, D), lambda i: (0, 0)),
      ],
      out_specs=pl.BlockSpec((bm, D), lambda i: (i, 0)),
      out_shape=jax.ShapeDtypeStruct((n_pad, D), jnp.float32),
  )


def kernel(adjacency_matrix, graph, W, b):
  depth = W.shape[0]
  n = graph.shape[1]
  align = NW * NB * 8
  n_pad = ((n + align - 1) // align) * align

  h = jnp.pad(graph[0], ((0, n_pad - n), (0, 0)))
  idx = jnp.pad(adjacency_matrix.T.astype(jnp.int32),
                ((0, n_pad - n), (0, 0))).reshape(NW, -1, G)
  idx = jnp.pad(idx, ((0, 0), (0, NBUF), (0, 0)))

  gsum = _make_gather_sum(n_pad)
  mm = _make_mm_relu(n_pad, 512)
  for t in range(1, depth):
    m = gsum(h, idx)
    h = mm(m, W[t], b[t].reshape(1, D))
  return h[:n][None]


# speedup vs baseline: 1.1815x; 1.1815x over previous
import functools

import jax
import jax.numpy as jnp
from jax import lax
from jax.experimental import pallas as pl
from jax.experimental.pallas import tpu as pltpu
from jax.experimental.pallas import tpu_sc as plsc

D = 128
K = 32
L = 16
NC, NS = 2, 16
NW = NC * NS
NB = 4
G = NB * K


NBUF = 2


def _make_gather_sum(n_pad):
  chunk = n_pad // NW
  nsub = chunk // NB
  mesh = plsc.VectorSubcoreMesh(core_axis_name="c", subcore_axis_name="s")

  @functools.partial(
      pl.kernel,
      mesh=mesh,
      out_type=jax.ShapeDtypeStruct((n_pad, D), jnp.float32),
      scratch_types=[
          pltpu.VMEM((nsub + NBUF, G), jnp.int32),
          pltpu.VMEM((NBUF, G, D), jnp.float32),
          pltpu.VMEM((NBUF * NB, D), jnp.float32),
          pltpu.VMEM_SHARED((n_pad, D), jnp.float32),
          pltpu.SemaphoreType.DMA,
          pltpu.SemaphoreType.DMA,
      ],
  )
  def gsum(h_hbm, idx_hbm, out_hbm, idx_v, gbuf, outv, sbuf, sem0, sem1):
    wid = lax.axis_index("c") * NS + lax.axis_index("s")
    sid = lax.axis_index("s")
    sems = (sem0, sem1)
    pltpu.sync_copy(idx_hbm.at[wid], idx_v)

    srows = n_pad // NS
    soff = sid * srows
    pltpu.sync_copy(h_hbm.at[pl.ds(soff, srows)], sbuf.at[pl.ds(soff, srows)])
    plsc.subcore_barrier()

    for b in range(NBUF):
      pltpu.async_copy(sbuf.at[idx_v.at[b]], gbuf.at[b], sems[b])

    def outer(g, carry):
      for b in range(NBUF):
        s = g * NBUF + b
        pltpu.make_async_copy(sbuf.at[idx_v.at[s]], gbuf.at[b],
                              sems[b]).wait()
        for n in range(NB):
          row = n * K
          vals = [[gbuf[b, row + r, pl.ds(j * L, L)] for j in range(D // L)]
                  for r in range(4)]
          for j in range(D // L):
            outv[b * NB + n, pl.ds(j * L, L)] = (
                (vals[0][j] + vals[1][j]) + (vals[2][j] + vals[3][j]))

          def kgroup(kg, carry, n=n, b=b):
            row = n * K + kg * 4
            vals = [[gbuf[b, row + r, pl.ds(j * L, L)] for j in range(D // L)]
                    for r in range(4)]
            for j in range(D // L):
              plsc.addupdate(outv.at[b * NB + n, pl.ds(j * L, L)],
                             (vals[0][j] + vals[1][j]) +
                             (vals[2][j] + vals[3][j]))
            return carry

          lax.fori_loop(1, K // 4, kgroup, 0)
        pltpu.async_copy(sbuf.at[idx_v.at[s + NBUF]], gbuf.at[b], sems[b])
      pltpu.sync_copy(outv,
                      out_hbm.at[pl.ds(wid * chunk + g * (NBUF * NB),
                                       NBUF * NB)])
      return carry

    lax.fori_loop(0, nsub // NBUF, outer, 0)
    for b in range(NBUF):
      pltpu.make_async_copy(sbuf.at[idx_v.at[b]], gbuf.at[b], sems[b]).wait()

  return gsum


def _make_mm_relu(n_pad, bm):
  def body(x_ref, w_ref, b_ref, o_ref):
    y = jnp.dot(x_ref[...], w_ref[...], preferred_element_type=jnp.float32)
    o_ref[...] = jnp.maximum(y * (1.0 / K) + b_ref[...], 0.0)

  return pl.pallas_call(
      body,
      grid=(n_pad // bm,),
      in_specs=[
          pl.BlockSpec((bm, D), lambda i: (i, 0)),
          pl.BlockSpec((D, D), lambda i: (0, 0)),
          pl.BlockSpec((1, D), lambda i: (0, 0)),
      ],
      out_specs=pl.BlockSpec((bm, D), lambda i: (i, 0)),
      out_shape=jax.ShapeDtypeStruct((n_pad, D), jnp.float32),
  )


def kernel(adjacency_matrix, graph, W, b):
  depth = W.shape[0]
  n = graph.shape[1]
  align = NW * NB * 8
  n_pad = ((n + align - 1) // align) * align

  h = jnp.pad(graph[0], ((0, n_pad - n), (0, 0)))
  idx = jnp.pad(adjacency_matrix.T.astype(jnp.int32),
                ((0, n_pad - n), (0, 0))).reshape(NW, -1, G)
  idx = jnp.pad(idx, ((0, 0), (0, NBUF), (0, 0)))

  gsum = _make_gather_sum(n_pad)
  mm = _make_mm_relu(n_pad, 512)
  for t in range(1, depth):
    m = gsum(h, idx)
    h = mm(m, W[t], b[t].reshape(1, D))
  return h[:n][None]
